# Initial kernel scaffold; baseline (speedup 1.0000x reference)
import functools

import jax
import jax.numpy as jnp
from jax import lax
from jax.experimental import pallas as pl
from jax.experimental.pallas import tpu as pltpu
from jax.experimental.pallas import tpu_sc as plsc

PAD = 0
BATCH = 4096
SEQ = 200
D = 128
PE_ROWS = 201
NC, NS, L = 2, 16, 16
NW = NC * NS
RPW = BATCH // NW
ROW_WORDS = SEQ * D
PE_WORDS = PE_ROWS * D

_mesh = plsc.VectorSubcoreMesh(core_axis_name="c", subcore_axis_name="s")


@functools.partial(
    pl.kernel,
    out_type=jax.ShapeDtypeStruct((BATCH * SEQ * D,), jnp.float32),
    mesh=_mesh,
    scratch_types=[
        pltpu.VMEM((PE_WORDS,), jnp.float32),
        pltpu.VMEM((RPW * SEQ,), jnp.int32),
        pltpu.VMEM((ROW_WORDS,), jnp.float32),
        pltpu.VMEM((ROW_WORDS,), jnp.float32),
        pltpu.VMEM((RPW,), jnp.int32),
    ],
)
def _pe_lookup(x_hbm, pe_hbm, out_hbm, pe_v, x_v, static_v, outbuf_v, flag_v):
    wid = lax.axis_index("s") * NC + lax.axis_index("c")
    base_row = wid * RPW

    pltpu.sync_copy(pe_hbm, pe_v)
    pltpu.sync_copy(pe_hbm.at[pl.ds(D, ROW_WORDS)], static_v)
    pltpu.sync_copy(x_hbm.at[pl.ds(base_row * SEQ, RPW * SEQ)], x_v)

    lane = lax.iota(jnp.int32, L)

    def scan_group(g, carry):
        row_addr = (g * L + lane) * SEQ

        def step(s, pos):
            v = plsc.load_gather(x_v, [row_addr + s])
            m = v != PAD
            pos = pos + m.astype(jnp.int32)
            plsc.store_scatter(x_v, [row_addr + s], jnp.where(m, pos, 0))
            return pos

        pos = lax.fori_loop(0, SEQ, step, jnp.zeros((L,), jnp.int32))
        plsc.store_scatter(flag_v, [g * L + lane], (pos == SEQ).astype(jnp.int32))
        return carry

    lax.fori_loop(0, RPW // L, scan_group, 0)

    def emit_row(b, carry):
        out_off = (base_row + b) * ROW_WORDS
        flag = flag_v[b]

        @pl.when(flag == 1)
        def _clean():
            pltpu.sync_copy(static_v, out_hbm.at[pl.ds(out_off, ROW_WORDS)])

        @pl.when(flag == 0)
        def _dirty():
            xb = b * SEQ
            for w in range(13):
                s0 = w * 16 if w < 12 else SEQ - 16
                idx = plsc.load_gather(x_v, [xb + s0 + lane])

                def dstep(d, carry2):
                    pe_addr, out_addr = carry2
                    vals = plsc.load_gather(pe_v, [pe_addr])
                    plsc.store_scatter(outbuf_v, [out_addr], vals)
                    return (pe_addr + 1, out_addr + 1)

                lax.fori_loop(0, D, dstep, (idx * D, (s0 + lane) * D))
            pltpu.sync_copy(outbuf_v, out_hbm.at[pl.ds(out_off, ROW_WORDS)])

        return carry

    lax.fori_loop(0, RPW, emit_row, 0)


def kernel(x, pe):
    xf = x.reshape(-1).astype(jnp.int32)
    pef = pe.reshape(-1).astype(jnp.float32)
    out = _pe_lookup(xf, pef)
    return out.reshape(BATCH, SEQ, D)


# speedup vs baseline: 12.4320x; 12.4320x over previous
import functools

import jax
import jax.numpy as jnp
from jax import lax
from jax.experimental import pallas as pl
from jax.experimental.pallas import tpu as pltpu
from jax.experimental.pallas import tpu_sc as plsc

PAD = 0
BATCH = 4096
SEQ = 200
D = 128
PE_ROWS = 201
NC, NS, L = 2, 16, 16
NW = NC * NS
RPW = BATCH // NW
ROW_WORDS = SEQ * D
PE_WORDS = PE_ROWS * D

_mesh = plsc.VectorSubcoreMesh(core_axis_name="c", subcore_axis_name="s")


@functools.partial(
    pl.kernel,
    out_type=jax.ShapeDtypeStruct((BATCH * SEQ * D,), jnp.float32),
    mesh=_mesh,
    compiler_params=pltpu.CompilerParams(needs_layout_passes=False),
    scratch_types=[
        pltpu.VMEM((PE_WORDS,), jnp.float32),
        pltpu.VMEM((RPW * SEQ,), jnp.int32),
        pltpu.VMEM((ROW_WORDS,), jnp.float32),
        pltpu.VMEM((ROW_WORDS,), jnp.float32),
        pltpu.VMEM((RPW + L,), jnp.int32),
    ],
)
def _pe_lookup(x_hbm, pe_hbm, out_hbm, pe_v, x_v, static_v, outbuf_v, flag_v):
    wid = lax.axis_index("s") * NC + lax.axis_index("c")
    base_row = wid * RPW

    pltpu.sync_copy(pe_hbm, pe_v)
    pltpu.sync_copy(pe_hbm.at[pl.ds(D, ROW_WORDS)], static_v)
    pltpu.sync_copy(x_hbm.at[pl.ds(base_row * SEQ, RPW * SEQ)], x_v)

    lane = lax.iota(jnp.int32, L)

    def scan_group(g, carry):
        row_addr = (g * L + lane) * SEQ

        def step(s, pos):
            v = plsc.load_gather(x_v, [row_addr + s])
            m = v != PAD
            pos = pos + m.astype(jnp.int32)
            plsc.store_scatter(x_v, [row_addr + s], jnp.where(m, pos, 0))
            return pos

        pos = lax.fori_loop(0, SEQ, step, jnp.zeros((L,), jnp.int32))
        plsc.store_scatter(flag_v, [g * L + lane], (pos == SEQ).astype(jnp.int32))
        return carry

    lax.fori_loop(0, RPW // L, scan_group, 0)

    def emit_row(b, carry):
        out_off = (base_row + b) * ROW_WORDS
        flag = flag_v[pl.ds(b, L)][0]

        @pl.when(flag == 1)
        def _clean():
            pltpu.sync_copy(static_v, out_hbm.at[pl.ds(out_off, ROW_WORDS)])

        @pl.when(flag == 0)
        def _dirty():
            xb = b * SEQ
            for w in range(13):
                s0 = w * 16 if w < 12 else SEQ - 16
                idx = plsc.load_gather(x_v, [xb + s0 + lane])

                def dstep(d, carry2):
                    pe_addr, out_addr = carry2
                    vals = plsc.load_gather(pe_v, [pe_addr])
                    plsc.store_scatter(outbuf_v, [out_addr], vals)
                    return (pe_addr + 1, out_addr + 1)

                lax.fori_loop(0, D, dstep, (idx * D, (s0 + lane) * D))
            pltpu.sync_copy(outbuf_v, out_hbm.at[pl.ds(out_off, ROW_WORDS)])

        return carry

    lax.fori_loop(0, RPW, emit_row, 0)


def kernel(x, pe):
    xf = x.reshape(-1).astype(jnp.int32)
    pef = pe.reshape(-1).astype(jnp.float32)
    out = _pe_lookup(xf, pef)
    return out.reshape(BATCH, SEQ, D)
